# baseline (speedup 1.0000x reference)
import functools

import jax
import jax.numpy as jnp
from jax import lax
from jax.experimental import pallas as pl
from jax.experimental.pallas import tpu as pltpu
from jax.experimental.pallas import tpu_sc as plsc

_B, _S, _DP, _DM, _SEQ = 4, 4096, 768, 1536, 4096
_M = _B * _S
_NSPLIT = 4
_MS = _M // _NSPLIT
_NW = 32
_RPW = _MS // _NW
_CHUNK = 8
_NCHUNK = _RPW // _CHUNK


def _gather_sum_body(idx_hbm, depth_hbm, row_hbm, col_hbm, out_hbm,
                     idx_v, b0, c0, o0, b1, c1, o1,
                     g0, g1, s0, s1):
    wid = lax.axis_index("s") * 2 + lax.axis_index("c")
    base = wid * _RPW
    pltpu.sync_copy(idx_hbm.at[:, wid], idx_v)

    sets = ((b0, c0, o0, g0, s0), (b1, c1, o1, g1, s1))

    def fire_bc(c, b, cc, g):
        pltpu.async_copy(row_hbm.at[idx_v.at[1, c]], b, g)
        pltpu.async_copy(col_hbm.at[idx_v.at[2, c]], cc, g)

    def fire_depth(c, o, g):
        pltpu.async_copy(depth_hbm.at[idx_v.at[0, c]], o, g)

    def wait3(b, cc, o, g):
        pltpu.make_async_copy(row_hbm.at[idx_v.at[1, 0]], b, g).wait()
        pltpu.make_async_copy(col_hbm.at[idx_v.at[2, 0]], cc, g).wait()
        pltpu.make_async_copy(depth_hbm.at[idx_v.at[0, 0]], o, g).wait()

    fire_bc(0, b0, c0, g0)
    fire_bc(1, b1, c1, g1)
    fire_depth(0, o0, g0)

    def outer(i, _):
        for p in range(2):
            b, cc, o, g, s = sets[p]
            bq, cq, oq, gq, sq = sets[1 - p]
            c = 2 * i + p

            @pl.when(c > 0)
            def _():
                pltpu.make_async_copy(oq, out_hbm.at[pl.ds(0, _CHUNK)],
                                      sq).wait()

            @pl.when(c + 1 < _NCHUNK)
            def _():
                fire_depth(c + 1, oq, gq)

            wait3(b, cc, o, g)

            @plsc.parallel_loop(0, _DM // 16, 1, unroll=4)
            def _col(j):
                sl = pl.ds(j * 16, 16)
                for r in range(_CHUNK):
                    plsc.addupdate(o.at[r, sl], b[r, sl] + cc[r, sl])
            pltpu.async_copy(o, out_hbm.at[pl.ds(base + c * _CHUNK, _CHUNK)],
                             s)

            @pl.when(c + 2 < _NCHUNK)
            def _():
                fire_bc(c + 2, b, cc, g)
        return 0

    lax.fori_loop(0, _NCHUNK // 2, outer, 0)
    pltpu.make_async_copy(o1, out_hbm.at[pl.ds(0, _CHUNK)], s1).wait()


_gather_sum = functools.partial(
    pl.kernel,
    out_type=jax.ShapeDtypeStruct((_MS, _DM), jnp.float32),
    mesh=plsc.VectorSubcoreMesh(core_axis_name="c", subcore_axis_name="s"),
    scratch_types=[
        pltpu.VMEM((3, _NCHUNK, _CHUNK), jnp.int32),
        pltpu.VMEM((_CHUNK, _DM), jnp.float32),
        pltpu.VMEM((_CHUNK, _DM), jnp.float32),
        pltpu.VMEM((_CHUNK, _DM), jnp.float32),
        pltpu.VMEM((_CHUNK, _DM), jnp.float32),
        pltpu.VMEM((_CHUNK, _DM), jnp.float32),
        pltpu.VMEM((_CHUNK, _DM), jnp.float32),
        pltpu.SemaphoreType.DMA,
        pltpu.SemaphoreType.DMA,
        pltpu.SemaphoreType.DMA,
        pltpu.SemaphoreType.DMA,
    ],
)(_gather_sum_body)


_TM = 512
_TPS = _MS // _TM


def _mm_body(fp_ref, w_ref, b_ref, g_ref, o_ref):
    x = fp_ref[:, 3:].astype(jnp.bfloat16)
    acc = jnp.dot(x, w_ref[...], preferred_element_type=jnp.float32)
    o_ref[...] = acc + b_ref[...] + g_ref[...]


def _mm_body_alias(prev_ref, fp_ref, w_ref, b_ref, g_ref, o_ref):
    _mm_body(fp_ref, w_ref, b_ref, g_ref, o_ref)


def _matmul_add(h, fp, w, b2d, g, prev):
    off = h * _TPS
    in_specs = [
        pl.BlockSpec((_TM, 3 + _DP), lambda i: (i + off, 0)),
        pl.BlockSpec((_DP, _DM), lambda i: (0, 0)),
        pl.BlockSpec((1, _DM), lambda i: (0, 0)),
        pl.BlockSpec((_TM, _DM), lambda i: (i, 0)),
    ]
    out_specs = pl.BlockSpec((_TM, _DM), lambda i: (i + off, 0))
    out_shape = jax.ShapeDtypeStruct((_M, _DM), jnp.float32)
    if prev is None:
        return pl.pallas_call(
            _mm_body, grid=(_TPS,), in_specs=in_specs,
            out_specs=out_specs, out_shape=out_shape,
        )(fp, w, b2d, g)
    in_specs = [pl.BlockSpec((8, 128), lambda i: (0, 0))] + in_specs
    return pl.pallas_call(
        _mm_body_alias, grid=(_TPS,), in_specs=in_specs,
        out_specs=out_specs, out_shape=out_shape,
        input_output_aliases={0: 0},
    )(prev, fp, w, b2d, g)


def kernel(flattened_patches, W, b, depth_table, row_table, col_table):
    fp = flattened_patches.reshape(_M, 3 + _DP)
    idx = fp[:, :3].astype(jnp.int32).T.reshape(
        3, _NSPLIT, _NW, _NCHUNK, _CHUNK)
    w = W.astype(jnp.bfloat16)
    b2d = b.reshape(1, _DM)
    gs = [_gather_sum(idx[:, h], depth_table, row_table, col_table)
          for h in range(_NSPLIT)]
    out = None
    for h in range(_NSPLIT):
        out = _matmul_add(h, fp, w, b2d, gs[h], out)
    return out.reshape(_B, _S, _DM)


# speedup vs baseline: 1.0221x; 1.0221x over previous
import functools

import jax
import jax.numpy as jnp
from jax import lax
from jax.experimental import pallas as pl
from jax.experimental.pallas import tpu as pltpu
from jax.experimental.pallas import tpu_sc as plsc

_B, _S, _DP, _DM, _SEQ = 4, 4096, 768, 1536, 4096
_M = _B * _S
_NSPLIT = 2
_MS = _M // _NSPLIT
_NW = 32
_RPW = _MS // _NW
_CHUNK = 8
_NCHUNK = _RPW // _CHUNK


def _gather_sum_body(idx_hbm, depth_hbm, row_hbm, col_hbm, out_hbm,
                     idx_v, b0, c0, o0, b1, c1, o1,
                     g0, g1, s0, s1):
    wid = lax.axis_index("s") * 2 + lax.axis_index("c")
    base = wid * _RPW
    pltpu.sync_copy(idx_hbm.at[:, wid], idx_v)

    sets = ((b0, c0, o0, g0, s0), (b1, c1, o1, g1, s1))

    def fire_bc(c, b, cc, g):
        pltpu.async_copy(row_hbm.at[idx_v.at[1, c]], b, g)
        pltpu.async_copy(col_hbm.at[idx_v.at[2, c]], cc, g)

    def fire_depth(c, o, g):
        pltpu.async_copy(depth_hbm.at[idx_v.at[0, c]], o, g)

    def wait3(b, cc, o, g):
        pltpu.make_async_copy(row_hbm.at[idx_v.at[1, 0]], b, g).wait()
        pltpu.make_async_copy(col_hbm.at[idx_v.at[2, 0]], cc, g).wait()
        pltpu.make_async_copy(depth_hbm.at[idx_v.at[0, 0]], o, g).wait()

    fire_bc(0, b0, c0, g0)
    fire_bc(1, b1, c1, g1)
    fire_depth(0, o0, g0)

    def outer(i, _):
        for p in range(2):
            b, cc, o, g, s = sets[p]
            bq, cq, oq, gq, sq = sets[1 - p]
            c = 2 * i + p

            @pl.when(c > 0)
            def _():
                pltpu.make_async_copy(oq, out_hbm.at[pl.ds(0, _CHUNK)],
                                      sq).wait()

            @pl.when(c + 1 < _NCHUNK)
            def _():
                fire_depth(c + 1, oq, gq)

            wait3(b, cc, o, g)

            @plsc.parallel_loop(0, _DM // 16, 1, unroll=8)
            def _col(j):
                sl = pl.ds(j * 16, 16)
                for r in range(_CHUNK):
                    plsc.addupdate(o.at[r, sl], b[r, sl] + cc[r, sl])
            pltpu.async_copy(o, out_hbm.at[pl.ds(base + c * _CHUNK, _CHUNK)],
                             s)

            @pl.when(c + 2 < _NCHUNK)
            def _():
                fire_bc(c + 2, b, cc, g)
        return 0

    lax.fori_loop(0, _NCHUNK // 2, outer, 0)
    pltpu.make_async_copy(o1, out_hbm.at[pl.ds(0, _CHUNK)], s1).wait()


_gather_sum = functools.partial(
    pl.kernel,
    out_type=jax.ShapeDtypeStruct((_MS, _DM), jnp.float32),
    mesh=plsc.VectorSubcoreMesh(core_axis_name="c", subcore_axis_name="s"),
    scratch_types=[
        pltpu.VMEM((3, _NCHUNK, _CHUNK), jnp.int32),
        pltpu.VMEM((_CHUNK, _DM), jnp.float32),
        pltpu.VMEM((_CHUNK, _DM), jnp.float32),
        pltpu.VMEM((_CHUNK, _DM), jnp.float32),
        pltpu.VMEM((_CHUNK, _DM), jnp.float32),
        pltpu.VMEM((_CHUNK, _DM), jnp.float32),
        pltpu.VMEM((_CHUNK, _DM), jnp.float32),
        pltpu.SemaphoreType.DMA,
        pltpu.SemaphoreType.DMA,
        pltpu.SemaphoreType.DMA,
        pltpu.SemaphoreType.DMA,
    ],
)(_gather_sum_body)


_TM = 512
_TPS = _MS // _TM


def _mm_body(fp_ref, w_ref, b_ref, g_ref, o_ref):
    x = fp_ref[:, 3:].astype(jnp.bfloat16)
    acc = jnp.dot(x, w_ref[...], preferred_element_type=jnp.float32)
    o_ref[...] = acc + b_ref[...] + g_ref[...]


def _mm_body_alias(prev_ref, fp_ref, w_ref, b_ref, g_ref, o_ref):
    _mm_body(fp_ref, w_ref, b_ref, g_ref, o_ref)


def _matmul_add(h, fp, w, b2d, g, prev):
    off = h * _TPS
    in_specs = [
        pl.BlockSpec((_TM, 3 + _DP), lambda i: (i + off, 0)),
        pl.BlockSpec((_DP, _DM), lambda i: (0, 0)),
        pl.BlockSpec((1, _DM), lambda i: (0, 0)),
        pl.BlockSpec((_TM, _DM), lambda i: (i, 0)),
    ]
    out_specs = pl.BlockSpec((_TM, _DM), lambda i: (i + off, 0))
    out_shape = jax.ShapeDtypeStruct((_M, _DM), jnp.float32)
    if prev is None:
        return pl.pallas_call(
            _mm_body, grid=(_TPS,), in_specs=in_specs,
            out_specs=out_specs, out_shape=out_shape,
        )(fp, w, b2d, g)
    in_specs = [pl.BlockSpec((8, 128), lambda i: (0, 0))] + in_specs
    return pl.pallas_call(
        _mm_body_alias, grid=(_TPS,), in_specs=in_specs,
        out_specs=out_specs, out_shape=out_shape,
        input_output_aliases={0: 0},
    )(prev, fp, w, b2d, g)


def kernel(flattened_patches, W, b, depth_table, row_table, col_table):
    fp = flattened_patches.reshape(_M, 3 + _DP)
    idx = fp[:, :3].astype(jnp.int32).T.reshape(
        3, _NSPLIT, _NW, _NCHUNK, _CHUNK)
    w = W.astype(jnp.bfloat16)
    b2d = b.reshape(1, _DM)
    gs = [_gather_sum(idx[:, h], depth_table, row_table, col_table)
          for h in range(_NSPLIT)]
    out = None
    for h in range(_NSPLIT):
        out = _matmul_add(h, fp, w, b2d, gs[h], out)
    return out.reshape(_B, _S, _DM)


# speedup vs baseline: 1.0340x; 1.0116x over previous
import functools

import jax
import jax.numpy as jnp
from jax import lax
from jax.experimental import pallas as pl
from jax.experimental.pallas import tpu as pltpu
from jax.experimental.pallas import tpu_sc as plsc

_B, _S, _DP, _DM, _SEQ = 4, 4096, 768, 1536, 4096
_M = _B * _S
_NSPLIT = 2
_MS = _M // _NSPLIT
_NW = 32
_RPW = _MS // _NW
_CHUNK = 8
_NCHUNK = _RPW // _CHUNK


def _gather_sum_body(idx_hbm, depth_hbm, row_hbm, col_hbm, out_hbm,
                     idx_v, b0, c0, o0, b1, c1, o1,
                     g0, g1, s0, s1):
    wid = lax.axis_index("s") * 2 + lax.axis_index("c")
    base = wid * _RPW
    pltpu.sync_copy(idx_hbm.at[:, wid], idx_v)

    sets = ((b0, c0, o0, g0, s0), (b1, c1, o1, g1, s1))

    def fire_bc(c, b, cc, g):
        pltpu.async_copy(row_hbm.at[idx_v.at[1, c]], b, g)
        pltpu.async_copy(col_hbm.at[idx_v.at[2, c]], cc, g)

    def fire_depth(c, o, g):
        pltpu.async_copy(depth_hbm.at[idx_v.at[0, c]], o, g)

    def wait3(b, cc, o, g):
        pltpu.make_async_copy(row_hbm.at[idx_v.at[1, 0]], b, g).wait()
        pltpu.make_async_copy(col_hbm.at[idx_v.at[2, 0]], cc, g).wait()
        pltpu.make_async_copy(depth_hbm.at[idx_v.at[0, 0]], o, g).wait()

    fire_bc(0, b0, c0, g0)
    fire_bc(1, b1, c1, g1)
    fire_depth(0, o0, g0)

    def outer(i, _):
        for p in range(2):
            b, cc, o, g, s = sets[p]
            bq, cq, oq, gq, sq = sets[1 - p]
            c = 2 * i + p

            @pl.when(c > 0)
            def _():
                pltpu.make_async_copy(oq, out_hbm.at[pl.ds(0, _CHUNK)],
                                      sq).wait()

            @pl.when(c + 1 < _NCHUNK)
            def _():
                fire_depth(c + 1, oq, gq)

            wait3(b, cc, o, g)

            @plsc.parallel_loop(0, _DM // 16, 1, unroll=8)
            def _col(j):
                sl = pl.ds(j * 16, 16)
                for r in range(_CHUNK):
                    plsc.addupdate(o.at[r, sl], b[r, sl] + cc[r, sl])
            pltpu.async_copy(o, out_hbm.at[pl.ds(base + c * _CHUNK, _CHUNK)],
                             s)

            @pl.when(c + 2 < _NCHUNK)
            def _():
                fire_bc(c + 2, b, cc, g)
        return 0

    lax.fori_loop(0, _NCHUNK // 2, outer, 0)
    pltpu.make_async_copy(o1, out_hbm.at[pl.ds(0, _CHUNK)], s1).wait()


_gather_sum = functools.partial(
    pl.kernel,
    out_type=jax.ShapeDtypeStruct((_MS, _DM), jnp.float32),
    mesh=plsc.VectorSubcoreMesh(core_axis_name="c", subcore_axis_name="s"),
    scratch_types=[
        pltpu.VMEM((3, _NCHUNK, _CHUNK), jnp.int32),
        pltpu.VMEM((_CHUNK, _DM), jnp.float32),
        pltpu.VMEM((_CHUNK, _DM), jnp.float32),
        pltpu.VMEM((_CHUNK, _DM), jnp.float32),
        pltpu.VMEM((_CHUNK, _DM), jnp.float32),
        pltpu.VMEM((_CHUNK, _DM), jnp.float32),
        pltpu.VMEM((_CHUNK, _DM), jnp.float32),
        pltpu.SemaphoreType.DMA,
        pltpu.SemaphoreType.DMA,
        pltpu.SemaphoreType.DMA,
        pltpu.SemaphoreType.DMA,
    ],
)(_gather_sum_body)


_TM = 1024
_TPS = _MS // _TM


def _mm_body(fp_ref, w_ref, b_ref, g_ref, o_ref):
    x = fp_ref[:, 3:].astype(jnp.bfloat16)
    acc = jnp.dot(x, w_ref[...], preferred_element_type=jnp.float32)
    o_ref[...] = acc + b_ref[...] + g_ref[...]


def _mm_body_alias(prev_ref, fp_ref, w_ref, b_ref, g_ref, o_ref):
    _mm_body(fp_ref, w_ref, b_ref, g_ref, o_ref)


def _matmul_add(h, fp, w, b2d, g, prev):
    off = h * _TPS
    in_specs = [
        pl.BlockSpec((_TM, 3 + _DP), lambda i: (i + off, 0)),
        pl.BlockSpec((_DP, _DM), lambda i: (0, 0)),
        pl.BlockSpec((1, _DM), lambda i: (0, 0)),
        pl.BlockSpec((_TM, _DM), lambda i: (i, 0)),
    ]
    out_specs = pl.BlockSpec((_TM, _DM), lambda i: (i + off, 0))
    out_shape = jax.ShapeDtypeStruct((_M, _DM), jnp.float32)
    if prev is None:
        return pl.pallas_call(
            _mm_body, grid=(_TPS,), in_specs=in_specs,
            out_specs=out_specs, out_shape=out_shape,
        )(fp, w, b2d, g)
    in_specs = [pl.BlockSpec((8, 128), lambda i: (0, 0))] + in_specs
    return pl.pallas_call(
        _mm_body_alias, grid=(_TPS,), in_specs=in_specs,
        out_specs=out_specs, out_shape=out_shape,
        input_output_aliases={0: 0},
    )(prev, fp, w, b2d, g)


def kernel(flattened_patches, W, b, depth_table, row_table, col_table):
    fp = flattened_patches.reshape(_M, 3 + _DP)
    idx = fp[:, :3].astype(jnp.int32).T.reshape(
        3, _NSPLIT, _NW, _NCHUNK, _CHUNK)
    w = W.astype(jnp.bfloat16)
    b2d = b.reshape(1, _DM)
    gs = [_gather_sum(idx[:, h], depth_table, row_table, col_table)
          for h in range(_NSPLIT)]
    out = None
    for h in range(_NSPLIT):
        out = _matmul_add(h, fp, w, b2d, gs[h], out)
    return out.reshape(_B, _S, _DM)
